# baseline (speedup 1.0000x reference)
import functools

import jax
import jax.numpy as jnp
from jax import lax
from jax.experimental import pallas as pl
from jax.experimental.pallas import tpu as pltpu
from jax.experimental.pallas import tpu_sc as plsc

LANES = 16
CHUNK = 128


def _vec_sqrt(x):
    xg = jnp.maximum(x, jnp.float32(1e-35))
    i = lax.bitcast_convert_type(xg, jnp.int32)
    i = jnp.int32(0x5F3759DF) - lax.shift_right_logical(i, jnp.int32(1))
    y = lax.bitcast_convert_type(i, jnp.float32)
    half = jnp.float32(0.5) * xg
    for _ in range(3):
        y = y * (jnp.float32(1.5) - half * y * y)
    return x * y


def _make_transe(B, D):
    info = plsc.get_sparse_core_info()
    NC, NS = info.num_cores, info.num_subcores
    NW = NC * NS
    per_w = B // NW
    n_chunks = per_w // CHUNK
    assert per_w % CHUNK == 0 and D % LANES == 0

    mesh = plsc.VectorSubcoreMesh(core_axis_name="c", subcore_axis_name="s")

    @functools.partial(
        pl.kernel,
        mesh=mesh,
        compiler_params=pltpu.CompilerParams(
            needs_layout_passes=False, use_tc_tiling_on_sc=False),
        out_type=(
            jax.ShapeDtypeStruct((B,), jnp.float32),
            jax.ShapeDtypeStruct((B,), jnp.float32),
        ),
        scratch_types=[
            pltpu.VMEM((CHUNK,), jnp.int32),
            pltpu.VMEM((CHUNK,), jnp.int32),
            pltpu.VMEM((CHUNK,), jnp.int32),
            pltpu.VMEM((CHUNK,), jnp.int32),
            pltpu.VMEM((CHUNK,), jnp.int32),
            pltpu.VMEM((CHUNK, D), jnp.float32),
            pltpu.VMEM((CHUNK, D), jnp.float32),
            pltpu.VMEM((CHUNK, D), jnp.float32),
            pltpu.VMEM((CHUNK, D), jnp.float32),
            pltpu.VMEM((CHUNK, D), jnp.float32),
            pltpu.VMEM((CHUNK,), jnp.float32),
            pltpu.VMEM((CHUNK,), jnp.float32),
            pltpu.SemaphoreType.DMA,
        ],
    )
    def transe(pos_idx, edge_type, neg_idx, ent, rel, pos_out, neg_out,
               ph_i, pt_i, nh_i, nt_i, r_i, ph, pt, nh, nt, rr, po, no, sem):
        wid = lax.axis_index("s") * NC + lax.axis_index("c")
        for chunk in range(n_chunks):
            base = wid * per_w + chunk * CHUNK
            pltpu.sync_copy(pos_idx.at[0, pl.ds(base, CHUNK)], ph_i)
            pltpu.sync_copy(pos_idx.at[1, pl.ds(base, CHUNK)], pt_i)
            pltpu.sync_copy(neg_idx.at[0, pl.ds(base, CHUNK)], nh_i)
            pltpu.sync_copy(neg_idx.at[1, pl.ds(base, CHUNK)], nt_i)
            pltpu.sync_copy(edge_type.at[pl.ds(base, CHUNK)], r_i)
            cps = [
                pltpu.async_copy(ent.at[ph_i], ph, sem),
                pltpu.async_copy(ent.at[pt_i], pt, sem),
                pltpu.async_copy(ent.at[nh_i], nh, sem),
                pltpu.async_copy(ent.at[nt_i], nt, sem),
                pltpu.async_copy(rel.at[r_i], rr, sem),
            ]
            for cp in cps:
                cp.wait()

            def group_body(g, _):
                row0 = g * LANES
                rows = row0 + lax.iota(jnp.int32, LANES)
                pacc = jnp.zeros((LANES,), jnp.float32)
                nacc = jnp.zeros((LANES,), jnp.float32)
                for d in range(D):
                    cols = jnp.full((LANES,), d, jnp.int32)
                    rv = plsc.load_gather(rr, [rows, cols])
                    pd = plsc.load_gather(ph, [rows, cols]) + rv \
                        - plsc.load_gather(pt, [rows, cols])
                    nd = plsc.load_gather(nh, [rows, cols]) + rv \
                        - plsc.load_gather(nt, [rows, cols])
                    pacc = pacc + pd * pd
                    nacc = nacc + nd * nd
                po[pl.ds(row0, LANES)] = _vec_sqrt(pacc)
                no[pl.ds(row0, LANES)] = _vec_sqrt(nacc)
                return 0

            lax.fori_loop(0, CHUNK // LANES, group_body, 0)

            pltpu.sync_copy(po, pos_out.at[pl.ds(base, CHUNK)])
            pltpu.sync_copy(no, neg_out.at[pl.ds(base, CHUNK)])

    return transe


def kernel(pos_edge_index, edge_type, neg_edge_index, entity_embeddings,
           relation_embeddings):
    B = pos_edge_index.shape[1]
    D = entity_embeddings.shape[1]
    fn = _make_transe(B, D)
    return fn(pos_edge_index, edge_type, neg_edge_index, entity_embeddings,
              relation_embeddings)


# speedup vs baseline: 1.1511x; 1.1511x over previous
import functools

import jax
import jax.numpy as jnp
from jax import lax
from jax.experimental import pallas as pl
from jax.experimental.pallas import tpu as pltpu
from jax.experimental.pallas import tpu_sc as plsc

LANES = 16
CHUNK = 128


def _vec_sqrt(x):
    xg = jnp.maximum(x, jnp.float32(1e-35))
    i = lax.bitcast_convert_type(xg, jnp.int32)
    i = jnp.int32(0x5F3759DF) - lax.shift_right_logical(i, jnp.int32(1))
    y = lax.bitcast_convert_type(i, jnp.float32)
    half = jnp.float32(0.5) * xg
    for _ in range(3):
        y = y * (jnp.float32(1.5) - half * y * y)
    return x * y


def _make_transe(B, D):
    info = plsc.get_sparse_core_info()
    NC, NS = info.num_cores, info.num_subcores
    NW = NC * NS
    per_w = B // NW
    n_chunks = per_w // CHUNK
    assert per_w % CHUNK == 0 and D % LANES == 0

    mesh = plsc.VectorSubcoreMesh(core_axis_name="c", subcore_axis_name="s")

    row_buf = pltpu.VMEM((CHUNK, D), jnp.float32)
    idx_buf = pltpu.VMEM((per_w,), jnp.int32)

    @functools.partial(
        pl.kernel,
        mesh=mesh,
        compiler_params=pltpu.CompilerParams(
            needs_layout_passes=False, use_tc_tiling_on_sc=False),
        out_type=(
            jax.ShapeDtypeStruct((B,), jnp.float32),
            jax.ShapeDtypeStruct((B,), jnp.float32),
        ),
        scratch_types=[
            idx_buf, idx_buf, idx_buf, idx_buf, idx_buf,
            row_buf, row_buf, row_buf, row_buf, row_buf,
            row_buf, row_buf, row_buf, row_buf, row_buf,
            pltpu.VMEM((per_w,), jnp.float32),
            pltpu.VMEM((per_w,), jnp.float32),
            pltpu.SemaphoreType.DMA,
            pltpu.SemaphoreType.DMA,
            pltpu.SemaphoreType.DMA,
        ],
    )
    def transe(pos_idx, edge_type, neg_idx, ent, rel, pos_out, neg_out,
               ph_i, pt_i, nh_i, nt_i, r_i,
               ph0, pt0, nh0, nt0, rr0,
               ph1, pt1, nh1, nt1, rr1,
               po, no, sem_i, sem0, sem1):
        wid = lax.axis_index("s") * NC + lax.axis_index("c")
        base_w = wid * per_w
        bufs = ((ph0, pt0, nh0, nt0, rr0), (ph1, pt1, nh1, nt1, rr1))
        sems = (sem0, sem1)

        idx_cps = [
            pltpu.async_copy(pos_idx.at[0, pl.ds(base_w, per_w)], ph_i, sem_i),
            pltpu.async_copy(pos_idx.at[1, pl.ds(base_w, per_w)], pt_i, sem_i),
            pltpu.async_copy(neg_idx.at[0, pl.ds(base_w, per_w)], nh_i, sem_i),
            pltpu.async_copy(neg_idx.at[1, pl.ds(base_w, per_w)], nt_i, sem_i),
            pltpu.async_copy(edge_type.at[pl.ds(base_w, per_w)], r_i, sem_i),
        ]
        for cp in idx_cps:
            cp.wait()

        def fire(c, par):
            sl = pl.ds(c * CHUNK, CHUNK)
            bph, bpt, bnh, bnt, brr = bufs[par]
            sem = sems[par]
            return [
                pltpu.async_copy(ent.at[ph_i.at[sl]], bph, sem),
                pltpu.async_copy(ent.at[pt_i.at[sl]], bpt, sem),
                pltpu.async_copy(ent.at[nh_i.at[sl]], bnh, sem),
                pltpu.async_copy(ent.at[nt_i.at[sl]], bnt, sem),
                pltpu.async_copy(rel.at[r_i.at[sl]], brr, sem),
            ]

        lane_ids = lax.iota(jnp.int32, LANES)
        in_flight = fire(0, 0)
        for c in range(n_chunks):
            par = c & 1
            for cp in in_flight:
                cp.wait()
            if c + 1 < n_chunks:
                in_flight = fire(c + 1, 1 - par)
            bph, bpt, bnh, bnt, brr = bufs[par]
            out0 = c * CHUNK

            def group_body(g, _):
                row0 = g * LANES
                pvec = jnp.zeros((LANES,), jnp.float32)
                nvec = jnp.zeros((LANES,), jnp.float32)
                for j in range(LANES):
                    r = row0 + j
                    pacc = jnp.zeros((LANES,), jnp.float32)
                    nacc = jnp.zeros((LANES,), jnp.float32)
                    for d in range(D // LANES):
                        sl = pl.ds(d * LANES, LANES)
                        rv = brr[r, sl]
                        pd = bph[r, sl] + rv - bpt[r, sl]
                        nd = bnh[r, sl] + rv - bnt[r, sl]
                        pacc = pacc + pd * pd
                        nacc = nacc + nd * nd
                    jmask = lane_ids == j
                    pvec = jnp.where(jmask, jnp.sum(pacc), pvec)
                    nvec = jnp.where(jmask, jnp.sum(nacc), nvec)
                po[pl.ds(out0 + row0, LANES)] = _vec_sqrt(pvec)
                no[pl.ds(out0 + row0, LANES)] = _vec_sqrt(nvec)
                return 0

            lax.fori_loop(0, CHUNK // LANES, group_body, 0)

        pltpu.sync_copy(po, pos_out.at[pl.ds(base_w, per_w)])
        pltpu.sync_copy(no, neg_out.at[pl.ds(base_w, per_w)])

    return transe


def kernel(pos_edge_index, edge_type, neg_edge_index, entity_embeddings,
           relation_embeddings):
    B = pos_edge_index.shape[1]
    D = entity_embeddings.shape[1]
    fn = _make_transe(B, D)
    return fn(pos_edge_index, edge_type, neg_edge_index, entity_embeddings,
              relation_embeddings)
